# Initial kernel scaffold; baseline (speedup 1.0000x reference)
import jax
import jax.numpy as jnp
from jax.experimental import pallas as pl


def kernel(inputs):
    raise NotImplementedError("write your pallas kernel here")


# speedup vs baseline: 28.9230x; 28.9230x over previous
import functools
import jax
import jax.numpy as jnp
from jax import lax
from jax.experimental import pallas as pl
from jax.experimental.pallas import tpu as pltpu
from jax.experimental.pallas import tpu_sc as plsc

_B = 4
_T = 8192
_C = 1024
_K = 8

_NC = 2
_NS = 16
_NW = _NC * _NS
_CPW = _C // (_NW // _B)
_LG = _CPW // 16
_TC = 256
_NCHUNK = _T // _TC


def _sc_body(in_hbm, out_hbm, buf, obuf, sem):
    wid = lax.axis_index("s") * _NC + lax.axis_index("c")
    b = wid // (_NW // _B)
    c0 = (wid % (_NW // _B)) * _CPW

    neg_inf = jnp.full((16,), -jnp.inf, dtype=jnp.float32)

    def chunk_body(chunk, state):
        t0 = chunk * _TC
        pltpu.sync_copy(
            in_hbm.at[b, pl.ds(t0, _TC), pl.ds(c0, _CPW)], buf)

        new_state = []
        for l in range(_LG):
            s = list(state[l])

            def row_body(t, s):
                s = list(s)
                v = buf[t, pl.ds(16 * l, 16)]
                for j in range(_K):
                    lo = jnp.minimum(s[j], v)
                    s[j] = jnp.maximum(s[j], v)
                    v = lo
                return tuple(s)

            s = lax.fori_loop(0, _TC, row_body, tuple(s))
            new_state.append(s)
        return tuple(new_state)

    init = tuple(tuple(neg_inf for _ in range(_K)) for _ in range(_LG))
    state = lax.fori_loop(0, _NCHUNK, chunk_body, init)

    for l in range(_LG):
        for j in range(_K):
            obuf[j, pl.ds(16 * l, 16)] = state[l][j]

    pltpu.sync_copy(obuf, out_hbm.at[b, :, pl.ds(c0, _CPW)])


@jax.jit
def _kmax_sc(inputs):
    mesh = plsc.VectorSubcoreMesh(
        core_axis_name="c", subcore_axis_name="s",
        num_cores=_NC, num_subcores=_NS)
    kern = pl.kernel(
        _sc_body,
        out_type=jax.ShapeDtypeStruct((_B, _K, _C), jnp.float32),
        mesh=mesh,
        scratch_types=[
            pltpu.VMEM((_TC, _CPW), jnp.float32),
            pltpu.VMEM((_K, _CPW), jnp.float32),
            pltpu.SemaphoreType.DMA,
        ],
    )
    return kern(inputs)


def kernel(inputs):
    out = _kmax_sc(inputs)
    return out.transpose(0, 2, 1).reshape(_B, _C * _K)
